# baseline (speedup 1.0000x reference)
import functools

import jax
import jax.numpy as jnp
from jax import lax
from jax.experimental import pallas as pl
from jax.experimental.pallas import tpu as pltpu
from jax.experimental.pallas import tpu_sc as plsc

M = 1_000_000
E = 2_000_000
B = 16384
D = 15
NC = 2
NS = 16
NW = NC * NS
L = 16

NB = 2 * B
WCOLS = 1792
MTAIL = 64

PRIO_N = 18 * WCOLS
HALF_WINS = 558
EREM = 128
EREM_BASE = 1116 * WCOLS

_mesh = plsc.VectorSubcoreMesh(core_axis_name="c", subcore_axis_name="s")


def _iota16():
  return lax.iota(jnp.int32, L)


@functools.partial(
    pl.kernel,
    out_type=(
        jax.ShapeDtypeStruct((D, M), jnp.float32),
        jax.ShapeDtypeStruct((D, E), jnp.float32),
        jax.ShapeDtypeStruct((M,), jnp.float32),
    ),
    mesh=_mesh,
    compiler_params=pltpu.CompilerParams(needs_layout_passes=False),
    scratch_types=[
        pltpu.VMEM((PRIO_N,), jnp.int32),
        pltpu.VMEM((D, WCOLS), jnp.float32),
        pltpu.VMEM((WCOLS,), jnp.float32),
        pltpu.VMEM((WCOLS,), jnp.int32),
        pltpu.VMEM((WCOLS,), jnp.int32),
        pltpu.VMEM((2048,), jnp.int32),
        pltpu.VMEM((L * D,), jnp.int32),
        pltpu.VMEM((L * D,), jnp.float32),
        pltpu.VMEM_SHARED((NB * D,), jnp.float32),
        pltpu.VMEM_SHARED((B * D,), jnp.float32),
        pltpu.SemaphoreType.DMA,
        pltpu.SemaphoreType.DMA,
    ],
)
def _sc_update(mem_t, edge_t, node_ids, node_feat, edge_ids, edge_feat,
               zeros_hbm, nm_t, ne_t, upd, prio, win, updw, wcol, wpos,
               idch, gidx, gval, nfeat_sh, efeat_sh, si, sg):
  wid = lax.axis_index("c") * NS + lax.axis_index("s")
  tid = lax.axis_index("s")

  nsl = NB * D // NS
  esl = B * D // NS
  pltpu.sync_copy(node_feat.at[pl.ds(tid * nsl, nsl)],
                  nfeat_sh.at[pl.ds(tid * nsl, nsl)])
  pltpu.sync_copy(edge_feat.at[pl.ds(tid * esl, esl)],
                  efeat_sh.at[pl.ds(tid * esl, esl)])
  plsc.subcore_barrier()

  nwin = jnp.where(wid < 14, 18, 17)
  bwin = jnp.where(wid < 14, 18 * wid, 252 + 17 * (wid - 14))

  def zero_prio():
    pltpu.sync_copy(zeros_hbm, prio)

  def build_prio(ids_hbm, n_upd, lo, rlen):
    for c in range(n_upd // 2048):
      pltpu.sync_copy(ids_hbm.at[pl.ds(c * 2048, 2048)], idch)
      def bb(j, _):
        for t in range(2):
          jj = 2 * j + t
          a = idch[pl.ds(jj * L, L)]
          rel = a - lo
          m = (rel >= 0) & (rel < rlen)
          relc = jnp.where(m, rel, 0)
          pos = c * 2048 + jj * L + _iota16() + 1
          plsc.store_scatter(prio, [relc], pos, mask=m)
        return 0
      lax.fori_loop(0, 64, bb, 0)

  def scan(loff, wlen, with_upd):
    def sc(j, offv):
      for t in range(2):
        jj = 2 * j + t
        pv = prio[pl.ds(loff + jj * L, L)]
        m = pv > 0
        off = offv[0]
        plsc.store_compressed(wcol.at[pl.ds(off, L)], jj * L + _iota16(),
                              mask=m)
        plsc.store_compressed(wpos.at[pl.ds(off, L)], pv, mask=m)
        if with_upd:
          updw[pl.ds(jj * L, L)] = jnp.where(m, 1.0, 0.0)
        offv = offv + plsc.all_reduce_population_count(m)
      return offv
    offv = lax.fori_loop(0, wlen // (2 * L), sc, jnp.zeros((L,), jnp.int32))
    return offv[0]

  def apply(nwinners, feat_sh, colbase):
    def group(g, _):
      mg = (g * L + _iota16()) < nwinners
      cols = wcol[pl.ds(g * L, L)]
      wp = wpos[pl.ds(g * L, L)]
      bidx = jnp.where(mg, (wp - 1) * D, 0)
      for k in range(D):
        gidx[pl.ds(k * L, L)] = bidx + k
      pltpu.async_copy(feat_sh.at[gidx], gval, sg).wait()
      colc = jnp.where(mg, colbase + cols, 0)
      for k in range(D):
        plsc.store_scatter(
            win, [jnp.full((L,), k, jnp.int32), colc],
            gval[pl.ds(k * L, L)], mask=mg)
      return 0
    lax.fori_loop(0, (nwinners + L - 1) // L, group, 0)

  def phase(src_t, dst_t, feat_sh, gbase_win, with_upd):
    def wbody(w, _):
      base = pl.multiple_of((gbase_win + bwin + w) * WCOLS, 128)
      pltpu.async_copy(src_t.at[:, pl.ds(base, WCOLS)], win, si)
      nwinners = scan(w * WCOLS, WCOLS, with_upd)
      pltpu.make_async_copy(src_t.at[:, pl.ds(0, WCOLS)], win, si).wait()
      apply(nwinners, feat_sh, 0)
      pltpu.sync_copy(win, dst_t.at[:, pl.ds(base, WCOLS)])
      if with_upd:
        pltpu.sync_copy(updw, upd.at[pl.ds(base, WCOLS)])
      return 0
    lax.fori_loop(0, nwin, wbody, 0)

  zero_prio()
  build_prio(node_ids, NB, bwin * WCOLS, nwin * WCOLS)
  phase(mem_t, nm_t, nfeat_sh, 0, True)

  for half in range(2):
    gbase = half * HALF_WINS
    rlen = nwin * WCOLS
    if half == 1:
      rlen = rlen + jnp.where(wid == 31, EREM, 0)
    zero_prio()
    build_prio(edge_ids, B, (gbase + bwin) * WCOLS, rlen)
    phase(edge_t, ne_t, efeat_sh, gbase, False)
    if half == 1:
      @pl.when(wid == 31)
      def _():
        pltpu.sync_copy(edge_t.at[:, pl.ds(EREM_BASE, EREM)],
                        win.at[:, pl.ds(0, EREM)])
        nwinners = scan(17 * WCOLS, EREM, False)
        apply(nwinners, efeat_sh, 0)
        pltpu.sync_copy(win.at[:, pl.ds(0, EREM)],
                        ne_t.at[:, pl.ds(EREM_BASE, EREM)])


def kernel(mem, edge_mem, src_feature, dst_feature, edge_feature, srcID,
           dstID, edge_pos):
  node_ids = jnp.concatenate(
      [srcID.astype(jnp.int32), dstID.astype(jnp.int32)])
  node_feat2 = jnp.concatenate([src_feature, dst_feature], axis=0)
  node_feat = node_feat2.reshape(NB * D)
  edge_ids = edge_pos.astype(jnp.int32)
  edge_feat = edge_feature.reshape(B * D)
  zeros_hbm = jnp.zeros((PRIO_N,), jnp.int32)

  nm_t, ne_t, upd = _sc_update(mem.T, edge_mem.T, node_ids, node_feat,
                               edge_ids, edge_feat, zeros_hbm)
  new_mem = nm_t.T
  new_edge_mem = ne_t.T

  tail0 = M - MTAIL
  rows = tail0 + jnp.arange(MTAIL, dtype=jnp.int32)
  pos = jnp.arange(1, NB + 1, dtype=jnp.int32)
  wpos = jnp.max(jnp.where(node_ids[None, :] == rows[:, None], pos[None, :],
                           0), axis=1)
  gathered = node_feat2[jnp.maximum(wpos - 1, 0)]
  tail_old = lax.slice(mem, (tail0, 0), (M, D))
  tail_new = jnp.where((wpos > 0)[:, None], gathered, tail_old)
  new_mem = lax.dynamic_update_slice(new_mem, tail_new, (tail0, 0))
  upd = lax.dynamic_update_slice(upd, (wpos > 0).astype(jnp.float32),
                                 (tail0,))
  return new_mem, new_edge_mem, upd
